# baseline (speedup 1.0000x reference)
import functools

import jax
import jax.numpy as jnp
from jax import lax
from jax.experimental import pallas as pl
from jax.experimental.pallas import tpu as pltpu
from jax.experimental.pallas import tpu_sc as plsc

N, E, F, HID1, HID2, NCLS, NG = 10000, 320000, 128, 64, 128, 40, 64
NPAD = 10240
NC, NS, L = 2, 16, 16
NW = NC * NS
EC = 128
NECH = E // EC
ECH_FULL = NECH // NW
ECH_REM = NECH % NW
VC = 80
NVCH = N // VC
VCH_FULL = NVCH // NW
VCH_REM = NVCH % NW

@functools.cache
def _mesh():
    return plsc.VectorSubcoreMesh(core_axis_name="c", subcore_axis_name="s",
                                  num_cores=NC, num_subcores=NS)


def _sc(out_type, scratch_types):
    def deco(body):
        @functools.cache
        def build():
            return pl.kernel(
                body, out_type, mesh=_mesh(), scratch_types=scratch_types,
                compiler_params=pltpu.CompilerParams(
                    needs_layout_passes=False, use_tc_tiling_on_sc=False))

        def call(*args):
            return build()(*args)

        return call

    return deco


def _wid():
    return lax.axis_index("s") * NC + lax.axis_index("c")


def _zero_1d(ref, nwords):
    z = jnp.zeros((L,), jnp.float32)

    def body(i, _):
        ref[pl.ds(i * L, L)] = z
        return 0

    lax.fori_loop(0, nwords // L, body, 0)


def _zero_2d(ref, rows, cols):
    z = jnp.zeros((L,), jnp.float32)

    def body(i, _):
        for j in range(cols // L):
            ref[i, pl.ds(j * L, L)] = z
        return 0

    lax.fori_loop(0, rows, body, 0)


@_sc(
    out_type=(
        jax.ShapeDtypeStruct((NC, NPAD), jnp.float32),
        jax.ShapeDtypeStruct((NC, NG), jnp.float32),
    ),
    scratch_types=[
        pltpu.VMEM_SHARED((NPAD,), jnp.float32),
        pltpu.VMEM_SHARED((NG,), jnp.float32),
        pltpu.VMEM((640,), jnp.float32),
        pltpu.VMEM((EC,), jnp.int32),
        pltpu.VMEM((EC,), jnp.float32),
        pltpu.VMEM((VC,), jnp.int32),
        pltpu.VMEM((VC,), jnp.float32),
    ],
)
def _sc_pre(src_h, ea_h, batch_h, deg_out, cnt_out, deg_acc, cnt_acc, zbuf,
            eidx, eval_, nidx, ones):
    c = lax.axis_index("c")
    s = lax.axis_index("s")
    w = _wid()

    _zero_1d(zbuf, 640)
    pltpu.sync_copy(zbuf, deg_acc.at[pl.ds(s * 640, 640)])

    @pl.when(s == 0)
    def _():
        pltpu.sync_copy(zbuf.at[pl.ds(0, NG)], cnt_acc)

    one = jnp.ones((L,), jnp.float32)
    for j in range(VC // L):
        ones[pl.ds(j * L, L)] = one

    plsc.subcore_barrier()

    def edge_body(t, _):
        base = (w + NW * t) * EC
        pltpu.sync_copy(src_h.at[pl.ds(base, EC)], eidx)
        pltpu.sync_copy(ea_h.at[pl.ds(base, EC)], eval_)
        pltpu.sync_copy(eval_, deg_acc.at[eidx], add=True)
        return 0

    lax.fori_loop(0, ECH_FULL, edge_body, 0)

    @pl.when(w < ECH_REM)
    def _():
        edge_body(ECH_FULL, 0)

    def node_body(t, _):
        base = (w + NW * t) * VC
        pltpu.sync_copy(batch_h.at[pl.ds(base, VC)], nidx)
        pltpu.sync_copy(ones, cnt_acc.at[nidx], add=True)
        return 0

    lax.fori_loop(0, VCH_FULL, node_body, 0)

    @pl.when(w < VCH_REM)
    def _():
        node_body(VCH_FULL, 0)

    plsc.subcore_barrier()
    pltpu.sync_copy(deg_acc.at[pl.ds(s * 640, 640)],
                    deg_out.at[c, pl.ds(s * 640, 640)])

    @pl.when(s == 0)
    def _():
        pltpu.sync_copy(cnt_acc, cnt_out.at[c])


@_sc(
    out_type=jax.ShapeDtypeStruct((NECH, EC), jnp.float32),
    scratch_types=[
        pltpu.VMEM((NPAD,), jnp.float32),
        pltpu.VMEM((ECH_FULL + 1, EC), jnp.int32),
        pltpu.VMEM((ECH_FULL + 1, EC), jnp.int32),
        pltpu.VMEM((ECH_FULL + 1, EC), jnp.float32),
        pltpu.VMEM((ECH_FULL + 1, EC), jnp.float32),
    ],
)
def _sc_w(dis_h, src_h, dst_h, ea_h, w_out, dis_v, sslab, dslab, easlab, wslab):
    w = _wid()
    row_off = w * ECH_FULL + jnp.minimum(w, ECH_REM)
    extra = w < ECH_REM
    pltpu.sync_copy(dis_h, dis_v)
    pltpu.sync_copy(src_h.at[pl.ds(row_off, ECH_FULL)],
                    sslab.at[pl.ds(0, ECH_FULL)])
    pltpu.sync_copy(dst_h.at[pl.ds(row_off, ECH_FULL)],
                    dslab.at[pl.ds(0, ECH_FULL)])
    pltpu.sync_copy(ea_h.at[pl.ds(row_off, ECH_FULL)],
                    easlab.at[pl.ds(0, ECH_FULL)])

    @pl.when(extra)
    def _():
        pltpu.sync_copy(src_h.at[pl.ds(row_off + ECH_FULL, 1)],
                        sslab.at[pl.ds(ECH_FULL, 1)])
        pltpu.sync_copy(dst_h.at[pl.ds(row_off + ECH_FULL, 1)],
                        dslab.at[pl.ds(ECH_FULL, 1)])
        pltpu.sync_copy(ea_h.at[pl.ds(row_off + ECH_FULL, 1)],
                        easlab.at[pl.ds(ECH_FULL, 1)])

    def body(t, _):
        for g in range(EC // L):
            isrc = sslab[t, pl.ds(g * L, L)]
            idst = dslab[t, pl.ds(g * L, L)]
            dsrc = plsc.load_gather(dis_v, [isrc])
            ddst = plsc.load_gather(dis_v, [idst])
            wslab[t, pl.ds(g * L, L)] = -(
                dsrc * easlab[t, pl.ds(g * L, L)] * ddst)
        return 0

    lax.fori_loop(0, ECH_FULL, body, 0)

    @pl.when(extra)
    def _():
        body(ECH_FULL, 0)

    pltpu.sync_copy(wslab.at[pl.ds(0, ECH_FULL)],
                    w_out.at[pl.ds(row_off, ECH_FULL)])

    @pl.when(extra)
    def _():
        pltpu.sync_copy(wslab.at[pl.ds(ECH_FULL, 1)],
                        w_out.at[pl.ds(row_off + ECH_FULL, 1)])


NBUF = 4
MAIN = (ECH_FULL // NBUF) * NBUF
TAIL = ECH_FULL - MAIN


@_sc(
    out_type=jax.ShapeDtypeStruct((NC, NPAD, HID1), jnp.float32),
    scratch_types=[
        pltpu.VMEM_SHARED((NPAD, HID1), jnp.float32),
        pltpu.VMEM((EC, HID1), jnp.float32),
        pltpu.VMEM((ECH_FULL + 1, EC), jnp.int32),
        pltpu.VMEM((ECH_FULL + 1, EC), jnp.int32),
        pltpu.VMEM((ECH_FULL + 1, EC), jnp.float32),
    ]
    + [pltpu.VMEM((EC, HID1), jnp.float32)] * NBUF
    + [pltpu.SemaphoreType.DMA] * (2 * NBUF),
)
def _sc_prop(h_h, src_h, dst_h, w_h, out, acc, zbuf, sslab, dslab, wslab,
             r0, r1, r2, r3, g0, g1, g2, g3, s0, s1, s2, s3):
    c = lax.axis_index("c")
    s = lax.axis_index("s")
    w = _wid()
    rows = [r0, r1, r2, r3]
    gsem = [g0, g1, g2, g3]
    ssem = [s0, s1, s2, s3]
    row_off = w * ECH_FULL + jnp.minimum(w, ECH_REM)
    extra = w < ECH_REM

    _zero_2d(zbuf, EC, HID1)
    for k in range(5):
        pltpu.sync_copy(zbuf, acc.at[pl.ds(s * 640 + k * EC, EC)])

    pltpu.sync_copy(src_h.at[pl.ds(row_off, ECH_FULL)],
                    sslab.at[pl.ds(0, ECH_FULL)])
    pltpu.sync_copy(dst_h.at[pl.ds(row_off, ECH_FULL)],
                    dslab.at[pl.ds(0, ECH_FULL)])
    pltpu.sync_copy(w_h.at[pl.ds(row_off, ECH_FULL)],
                    wslab.at[pl.ds(0, ECH_FULL)])

    @pl.when(extra)
    def _():
        pltpu.sync_copy(src_h.at[pl.ds(row_off + ECH_FULL, 1)],
                        sslab.at[pl.ds(ECH_FULL, 1)])
        pltpu.sync_copy(dst_h.at[pl.ds(row_off + ECH_FULL, 1)],
                        dslab.at[pl.ds(ECH_FULL, 1)])
        pltpu.sync_copy(w_h.at[pl.ds(row_off + ECH_FULL, 1)],
                        wslab.at[pl.ds(ECH_FULL, 1)])

    plsc.subcore_barrier()

    def wait_scatter(b):
        pltpu.make_async_copy(rows[b], acc.at[dslab.at[0]], ssem[b]).wait()

    def mul(t, rbuf):
        def mbody(i, _):
            sp = plsc.load_gather(
                wslab, [jnp.full((L,), t, jnp.int32),
                        jnp.full((L,), i, jnp.int32)])
            for j in range(HID1 // L):
                rbuf[i, pl.ds(j * L, L)] = rbuf[i, pl.ds(j * L, L)] * sp
            return 0

        lax.fori_loop(0, EC, mbody, 0, unroll=8)

    def group(k, _):
        descs = []
        for b in range(NBUF):
            t = NBUF * k + b

            @pl.when(k > 0)
            def _(b=b):
                wait_scatter(b)

            descs.append(
                pltpu.async_copy(h_h.at[sslab.at[t]], rows[b], gsem[b]))
        for b in range(NBUF):
            t = NBUF * k + b
            descs[b].wait()
            mul(t, rows[b])
            pltpu.async_copy(rows[b], acc.at[dslab.at[t]], ssem[b], add=True)
        return 0

    lax.fori_loop(0, MAIN // NBUF, group, 0)

    for b in range(TAIL):
        wait_scatter(b)
        pltpu.async_copy(h_h.at[sslab.at[MAIN + b]], rows[b], gsem[b])

    @pl.when(extra)
    def _():
        wait_scatter(TAIL)
        pltpu.async_copy(h_h.at[sslab.at[ECH_FULL]], rows[TAIL], gsem[TAIL])

    for b in range(TAIL):
        pltpu.make_async_copy(h_h.at[sslab.at[MAIN + b]], rows[b],
                              gsem[b]).wait()
        mul(MAIN + b, rows[b])
        pltpu.async_copy(rows[b], acc.at[dslab.at[MAIN + b]], ssem[b],
                         add=True)

    @pl.when(extra)
    def _():
        pltpu.make_async_copy(h_h.at[sslab.at[ECH_FULL]], rows[TAIL],
                              gsem[TAIL]).wait()
        mul(ECH_FULL, rows[TAIL])
        pltpu.async_copy(rows[TAIL], acc.at[dslab.at[ECH_FULL]], ssem[TAIL],
                         add=True)

    for b in range(NBUF):
        wait_scatter(b)

    plsc.subcore_barrier()
    pltpu.sync_copy(acc.at[pl.ds(s * 640, 640)], out.at[c, pl.ds(s * 640, 640)])


@_sc(
    out_type=jax.ShapeDtypeStruct((NC, NG, HID2), jnp.float32),
    scratch_types=[
        pltpu.VMEM_SHARED((NG, HID2), jnp.float32),
        pltpu.VMEM((NG, HID2), jnp.float32),
        pltpu.VMEM((VC,), jnp.int32),
        pltpu.VMEM((VC, HID2), jnp.float32),
    ],
)
def _sc_pool(h_h, batch_h, out, acc, zbuf, nidx, rows):
    c = lax.axis_index("c")
    s = lax.axis_index("s")
    w = _wid()

    @pl.when(s == 0)
    def _():
        _zero_2d(zbuf, NG, HID2)
        pltpu.sync_copy(zbuf, acc)

    plsc.subcore_barrier()

    def body(t, _):
        base = (w + NW * t) * VC
        pltpu.sync_copy(batch_h.at[pl.ds(base, VC)], nidx)
        pltpu.sync_copy(h_h.at[pl.ds(base, VC)], rows)
        pltpu.sync_copy(rows, acc.at[nidx], add=True)
        return 0

    lax.fori_loop(0, VCH_FULL, body, 0)

    @pl.when(w < VCH_REM)
    def _():
        body(VCH_FULL, 0)

    plsc.subcore_barrier()

    @pl.when(s == 0)
    def _():
        pltpu.sync_copy(acc, out.at[c])


BR = 2000
GRID = N // BR


def _rows(cols):
    return pl.BlockSpec((BR, cols), lambda i: (i, 0))


def _prows(cols):
    return pl.BlockSpec((NC, BR, cols), lambda i: (0, i, 0))


def _full(shape):
    return pl.BlockSpec(shape, lambda i: (0,) * len(shape))


def _dot(a, b):
    return jnp.dot(a, b, preferred_element_type=jnp.float32)


def _tc_dis(deg_p):
    def body(dp, o):
        deg = dp[0] + dp[1]
        o[...] = jnp.where(deg > 0, lax.rsqrt(deg), 0.0)

    return pl.pallas_call(
        body,
        out_shape=jax.ShapeDtypeStruct((NPAD,), jnp.float32),
    )(deg_p)


def _tc_l1pre(x, W1):
    def body(xr, wr, a0, a1, a2):
        a0[...] = _dot(xr[...], wr[0])
        a1[...] = _dot(xr[...], wr[1])
        a2[...] = _dot(xr[...], wr[2])

    o = jax.ShapeDtypeStruct((N, HID1), jnp.float32)
    return pl.pallas_call(
        body,
        grid=(GRID,),
        in_specs=[_rows(F), _full((3, F, HID1))],
        out_specs=[_rows(HID1)] * 3,
        out_shape=[o, o, o],
    )(x, W1)


def _tc_comb(a, p, scale):
    def body(ar, pr, o):
        o[...] = ar[...] + scale * (pr[0] + pr[1])

    return pl.pallas_call(
        body,
        grid=(GRID,),
        in_specs=[_rows(a.shape[1]), _prows(a.shape[1])],
        out_specs=_rows(a.shape[1]),
        out_shape=jax.ShapeDtypeStruct(a.shape, jnp.float32),
    )(a, p)


def _tc_l1post(a0, a2, q, b1):
    def body(a0r, a2r, qr, br, o):
        o[...] = jnp.maximum(a0r[...] - a2r[...] + qr[0] + qr[1] + br[...], 0.0)

    return pl.pallas_call(
        body,
        grid=(GRID,),
        in_specs=[_rows(HID1), _rows(HID1), _prows(HID1), _full((HID1,))],
        out_specs=_rows(HID1),
        out_shape=jax.ShapeDtypeStruct((N, HID1), jnp.float32),
    )(a0, a2, q, b1)


def _tc_mid(h, p, W):
    dout = W.shape[2]

    def body(hr, pr, wr, t1, z):
        t = pr[0] + pr[1]
        t1[...] = t
        z[...] = _dot(hr[...], wr[0]) + _dot(t, wr[1])

    return pl.pallas_call(
        body,
        grid=(GRID,),
        in_specs=[_rows(HID1), _prows(HID1), _full(W.shape)],
        out_specs=[_rows(HID1), _rows(dout)],
        out_shape=[
            jax.ShapeDtypeStruct((N, HID1), jnp.float32),
            jax.ShapeDtypeStruct((N, dout), jnp.float32),
        ],
    )(h, p, W)


def _tc_post(z, q, h, W2k, b):
    dout = W2k.shape[1]

    def body(zr, qr, hr, wr, br, o):
        u = 2.0 * (qr[0] + qr[1]) - hr[...]
        o[...] = jnp.maximum(zr[...] + _dot(u, wr[...]) + br[...], 0.0)

    return pl.pallas_call(
        body,
        grid=(GRID,),
        in_specs=[_rows(dout), _prows(HID1), _rows(HID1), _full(W2k.shape),
                  _full(b.shape)],
        out_specs=_rows(dout),
        out_shape=jax.ShapeDtypeStruct((N, dout), jnp.float32),
    )(z, q, h, W2k, b)


def _tc_head(sums, cnt_p, Wl, bl):
    def body(sr, cr, wr, br, o):
        cnt = jnp.maximum(cr[0] + cr[1], 1.0)
        pooled = (sr[0] + sr[1]) / cnt[:, None]
        logits = _dot(pooled, wr[...]) + br[...]
        m = jnp.max(logits, axis=1, keepdims=True)
        lse = m + jnp.log(jnp.sum(jnp.exp(logits - m), axis=1, keepdims=True))
        o[...] = logits - lse

    return pl.pallas_call(
        body,
        out_shape=jax.ShapeDtypeStruct((NG, NCLS), jnp.float32),
    )(sums, cnt_p, Wl, bl)


def kernel(x, edge_index, edge_attr, batch, W1, b1, W2, b2, W3, b3, Wl, bl):
    src = edge_index[0]
    dst = edge_index[1]
    src2 = src.reshape(NECH, EC)
    dst2 = dst.reshape(NECH, EC)

    deg_p, cnt_p = _sc_pre(src, edge_attr, batch)
    dis = _tc_dis(deg_p)
    w = _sc_w(dis, src2, dst2, edge_attr.reshape(NECH, EC))

    a0, a1, a2 = _tc_l1pre(x, W1)
    p = _sc_prop(a2, src2, dst2, w)
    y = _tc_comb(a1, p, 2.0)
    q = _sc_prop(y, src2, dst2, w)
    h1 = _tc_l1post(a0, a2, q, b1)

    p = _sc_prop(h1, src2, dst2, w)
    t1, z = _tc_mid(h1, p, W2)
    q = _sc_prop(t1, src2, dst2, w)
    h2 = _tc_post(z, q, h1, W2[2], b2)

    p = _sc_prop(h2, src2, dst2, w)
    t1, z = _tc_mid(h2, p, W3)
    q = _sc_prop(t1, src2, dst2, w)
    h3 = _tc_post(z, q, h2, W3[2], b3)

    sums = _sc_pool(h3, batch)
    return _tc_head(sums, cnt_p, Wl, bl)


# speedup vs baseline: 1.0799x; 1.0799x over previous
import functools

import jax
import jax.numpy as jnp
from jax import lax
from jax.experimental import pallas as pl
from jax.experimental.pallas import tpu as pltpu
from jax.experimental.pallas import tpu_sc as plsc

N, E, F, HID1, HID2, NCLS, NG = 10000, 320000, 128, 64, 128, 40, 64
NPAD = 10240
NC, NS, L = 2, 16, 16
NW = NC * NS
EC = 128
NECH = E // EC
ECH_FULL = NECH // NW
ECH_REM = NECH % NW
VC = 80
NVCH = N // VC
VCH_FULL = NVCH // NW
VCH_REM = NVCH % NW

@functools.cache
def _mesh():
    return plsc.VectorSubcoreMesh(core_axis_name="c", subcore_axis_name="s",
                                  num_cores=NC, num_subcores=NS)


def _sc(out_type, scratch_types):
    def deco(body):
        @functools.cache
        def build():
            return pl.kernel(
                body, out_type, mesh=_mesh(), scratch_types=scratch_types,
                compiler_params=pltpu.CompilerParams(
                    needs_layout_passes=False, use_tc_tiling_on_sc=False))

        def call(*args):
            return build()(*args)

        return call

    return deco


def _wid():
    return lax.axis_index("s") * NC + lax.axis_index("c")


def _zero_1d(ref, nwords):
    z = jnp.zeros((L,), jnp.float32)

    def body(i, _):
        ref[pl.ds(i * L, L)] = z
        return 0

    lax.fori_loop(0, nwords // L, body, 0)


def _zero_2d(ref, rows, cols):
    z = jnp.zeros((L,), jnp.float32)

    def body(i, _):
        for j in range(cols // L):
            ref[i, pl.ds(j * L, L)] = z
        return 0

    lax.fori_loop(0, rows, body, 0)


FIRE = 8
EGRP = ECH_FULL // FIRE
ETAIL = ECH_FULL - EGRP * FIRE


@_sc(
    out_type=(
        jax.ShapeDtypeStruct((NC, NPAD), jnp.float32),
        jax.ShapeDtypeStruct((NC, NG), jnp.float32),
    ),
    scratch_types=[
        pltpu.VMEM_SHARED((NPAD,), jnp.float32),
        pltpu.VMEM_SHARED((NG,), jnp.float32),
        pltpu.VMEM((640,), jnp.float32),
        pltpu.VMEM((ECH_FULL + 1, EC), jnp.int32),
        pltpu.VMEM((ECH_FULL + 1, EC), jnp.float32),
        pltpu.VMEM((VCH_FULL + 1, VC), jnp.int32),
        pltpu.VMEM((VC,), jnp.float32),
        pltpu.SemaphoreType.DMA,
    ],
)
def _sc_pre(src_h, ea_h, batch_h, deg_out, cnt_out, deg_acc, cnt_acc, zbuf,
            sslab, easlab, bslab, ones, sem):
    c = lax.axis_index("c")
    s = lax.axis_index("s")
    w = _wid()
    row_off = w * ECH_FULL + jnp.minimum(w, ECH_REM)
    extra = w < ECH_REM
    voff = w * VCH_FULL + jnp.minimum(w, VCH_REM)
    vextra = w < VCH_REM

    _zero_1d(zbuf, 640)
    pltpu.sync_copy(zbuf, deg_acc.at[pl.ds(s * 640, 640)])

    @pl.when(s == 0)
    def _():
        pltpu.sync_copy(zbuf.at[pl.ds(0, NG)], cnt_acc)

    one = jnp.ones((L,), jnp.float32)
    for j in range(VC // L):
        ones[pl.ds(j * L, L)] = one

    pltpu.sync_copy(src_h.at[pl.ds(row_off, ECH_FULL)],
                    sslab.at[pl.ds(0, ECH_FULL)])
    pltpu.sync_copy(ea_h.at[pl.ds(row_off, ECH_FULL)],
                    easlab.at[pl.ds(0, ECH_FULL)])
    pltpu.sync_copy(batch_h.at[pl.ds(voff, VCH_FULL)],
                    bslab.at[pl.ds(0, VCH_FULL)])

    @pl.when(extra)
    def _():
        pltpu.sync_copy(src_h.at[pl.ds(row_off + ECH_FULL, 1)],
                        sslab.at[pl.ds(ECH_FULL, 1)])
        pltpu.sync_copy(ea_h.at[pl.ds(row_off + ECH_FULL, 1)],
                        easlab.at[pl.ds(ECH_FULL, 1)])

    @pl.when(vextra)
    def _():
        pltpu.sync_copy(batch_h.at[pl.ds(voff + VCH_FULL, 1)],
                        bslab.at[pl.ds(VCH_FULL, 1)])

    plsc.subcore_barrier()

    def fire(t):
        return pltpu.async_copy(easlab.at[t], deg_acc.at[sslab.at[t]], sem,
                                add=True)

    def group(k, _):
        descs = [fire(FIRE * k + b) for b in range(FIRE)]
        for d in descs:
            d.wait()
        return 0

    lax.fori_loop(0, EGRP, group, 0)
    descs = [fire(EGRP * FIRE + b) for b in range(ETAIL)]
    for d in descs:
        d.wait()

    @pl.when(extra)
    def _():
        fire(ECH_FULL).wait()

    def vfire(t):
        return pltpu.async_copy(ones, cnt_acc.at[bslab.at[t]], sem, add=True)

    descs = [vfire(t) for t in range(VCH_FULL)]
    for d in descs:
        d.wait()

    @pl.when(vextra)
    def _():
        vfire(VCH_FULL).wait()

    plsc.subcore_barrier()
    pltpu.sync_copy(deg_acc.at[pl.ds(s * 640, 640)],
                    deg_out.at[c, pl.ds(s * 640, 640)])

    @pl.when(s == 0)
    def _():
        pltpu.sync_copy(cnt_acc, cnt_out.at[c])


@_sc(
    out_type=jax.ShapeDtypeStruct((NECH, EC), jnp.float32),
    scratch_types=[
        pltpu.VMEM((NPAD,), jnp.float32),
        pltpu.VMEM((ECH_FULL + 1, EC), jnp.int32),
        pltpu.VMEM((ECH_FULL + 1, EC), jnp.int32),
        pltpu.VMEM((ECH_FULL + 1, EC), jnp.float32),
        pltpu.VMEM((ECH_FULL + 1, EC), jnp.float32),
    ],
)
def _sc_w(dis_h, src_h, dst_h, ea_h, w_out, dis_v, sslab, dslab, easlab, wslab):
    w = _wid()
    row_off = w * ECH_FULL + jnp.minimum(w, ECH_REM)
    extra = w < ECH_REM
    pltpu.sync_copy(dis_h, dis_v)
    pltpu.sync_copy(src_h.at[pl.ds(row_off, ECH_FULL)],
                    sslab.at[pl.ds(0, ECH_FULL)])
    pltpu.sync_copy(dst_h.at[pl.ds(row_off, ECH_FULL)],
                    dslab.at[pl.ds(0, ECH_FULL)])
    pltpu.sync_copy(ea_h.at[pl.ds(row_off, ECH_FULL)],
                    easlab.at[pl.ds(0, ECH_FULL)])

    @pl.when(extra)
    def _():
        pltpu.sync_copy(src_h.at[pl.ds(row_off + ECH_FULL, 1)],
                        sslab.at[pl.ds(ECH_FULL, 1)])
        pltpu.sync_copy(dst_h.at[pl.ds(row_off + ECH_FULL, 1)],
                        dslab.at[pl.ds(ECH_FULL, 1)])
        pltpu.sync_copy(ea_h.at[pl.ds(row_off + ECH_FULL, 1)],
                        easlab.at[pl.ds(ECH_FULL, 1)])

    def body(t, _):
        for g in range(EC // L):
            isrc = sslab[t, pl.ds(g * L, L)]
            idst = dslab[t, pl.ds(g * L, L)]
            dsrc = plsc.load_gather(dis_v, [isrc])
            ddst = plsc.load_gather(dis_v, [idst])
            wslab[t, pl.ds(g * L, L)] = -(
                dsrc * easlab[t, pl.ds(g * L, L)] * ddst)
        return 0

    lax.fori_loop(0, ECH_FULL, body, 0)

    @pl.when(extra)
    def _():
        body(ECH_FULL, 0)

    pltpu.sync_copy(wslab.at[pl.ds(0, ECH_FULL)],
                    w_out.at[pl.ds(row_off, ECH_FULL)])

    @pl.when(extra)
    def _():
        pltpu.sync_copy(wslab.at[pl.ds(ECH_FULL, 1)],
                        w_out.at[pl.ds(row_off + ECH_FULL, 1)])


NBUF = 4
MAIN = (ECH_FULL // NBUF) * NBUF
TAIL = ECH_FULL - MAIN


@_sc(
    out_type=jax.ShapeDtypeStruct((NC, NPAD, HID1), jnp.float32),
    scratch_types=[
        pltpu.VMEM_SHARED((NPAD, HID1), jnp.float32),
        pltpu.VMEM((EC, HID1), jnp.float32),
        pltpu.VMEM((ECH_FULL + 1, EC), jnp.int32),
        pltpu.VMEM((ECH_FULL + 1, EC), jnp.int32),
        pltpu.VMEM((ECH_FULL + 1, EC), jnp.float32),
    ]
    + [pltpu.VMEM((EC, HID1), jnp.float32)] * NBUF
    + [pltpu.SemaphoreType.DMA] * (2 * NBUF),
)
def _sc_prop(h_h, src_h, dst_h, w_h, out, acc, zbuf, sslab, dslab, wslab,
             r0, r1, r2, r3, g0, g1, g2, g3, s0, s1, s2, s3):
    c = lax.axis_index("c")
    s = lax.axis_index("s")
    w = _wid()
    rows = [r0, r1, r2, r3]
    gsem = [g0, g1, g2, g3]
    ssem = [s0, s1, s2, s3]
    row_off = w * ECH_FULL + jnp.minimum(w, ECH_REM)
    extra = w < ECH_REM

    _zero_2d(zbuf, EC, HID1)
    for k in range(5):
        pltpu.sync_copy(zbuf, acc.at[pl.ds(s * 640 + k * EC, EC)])

    pltpu.sync_copy(src_h.at[pl.ds(row_off, ECH_FULL)],
                    sslab.at[pl.ds(0, ECH_FULL)])
    pltpu.sync_copy(dst_h.at[pl.ds(row_off, ECH_FULL)],
                    dslab.at[pl.ds(0, ECH_FULL)])
    pltpu.sync_copy(w_h.at[pl.ds(row_off, ECH_FULL)],
                    wslab.at[pl.ds(0, ECH_FULL)])

    @pl.when(extra)
    def _():
        pltpu.sync_copy(src_h.at[pl.ds(row_off + ECH_FULL, 1)],
                        sslab.at[pl.ds(ECH_FULL, 1)])
        pltpu.sync_copy(dst_h.at[pl.ds(row_off + ECH_FULL, 1)],
                        dslab.at[pl.ds(ECH_FULL, 1)])
        pltpu.sync_copy(w_h.at[pl.ds(row_off + ECH_FULL, 1)],
                        wslab.at[pl.ds(ECH_FULL, 1)])

    plsc.subcore_barrier()

    def wait_scatter(b):
        pltpu.make_async_copy(rows[b], acc.at[dslab.at[0]], ssem[b]).wait()

    def mul(t, rbuf):
        def mbody(i, _):
            sp = plsc.load_gather(
                wslab, [jnp.full((L,), t, jnp.int32),
                        jnp.full((L,), i, jnp.int32)])
            for j in range(HID1 // L):
                rbuf[i, pl.ds(j * L, L)] = rbuf[i, pl.ds(j * L, L)] * sp
            return 0

        lax.fori_loop(0, EC, mbody, 0, unroll=8)

    def group(k, _):
        descs = []
        for b in range(NBUF):
            t = NBUF * k + b

            @pl.when(k > 0)
            def _(b=b):
                wait_scatter(b)

            descs.append(
                pltpu.async_copy(h_h.at[sslab.at[t]], rows[b], gsem[b]))
        for b in range(NBUF):
            t = NBUF * k + b
            descs[b].wait()
            mul(t, rows[b])
            pltpu.async_copy(rows[b], acc.at[dslab.at[t]], ssem[b], add=True)
        return 0

    lax.fori_loop(0, MAIN // NBUF, group, 0)

    for b in range(TAIL):
        wait_scatter(b)
        pltpu.async_copy(h_h.at[sslab.at[MAIN + b]], rows[b], gsem[b])

    @pl.when(extra)
    def _():
        wait_scatter(TAIL)
        pltpu.async_copy(h_h.at[sslab.at[ECH_FULL]], rows[TAIL], gsem[TAIL])

    for b in range(TAIL):
        pltpu.make_async_copy(h_h.at[sslab.at[MAIN + b]], rows[b],
                              gsem[b]).wait()
        mul(MAIN + b, rows[b])
        pltpu.async_copy(rows[b], acc.at[dslab.at[MAIN + b]], ssem[b],
                         add=True)

    @pl.when(extra)
    def _():
        pltpu.make_async_copy(h_h.at[sslab.at[ECH_FULL]], rows[TAIL],
                              gsem[TAIL]).wait()
        mul(ECH_FULL, rows[TAIL])
        pltpu.async_copy(rows[TAIL], acc.at[dslab.at[ECH_FULL]], ssem[TAIL],
                         add=True)

    for b in range(NBUF):
        wait_scatter(b)

    plsc.subcore_barrier()
    pltpu.sync_copy(acc.at[pl.ds(s * 640, 640)], out.at[c, pl.ds(s * 640, 640)])


@_sc(
    out_type=jax.ShapeDtypeStruct((NC, NG, HID2), jnp.float32),
    scratch_types=[
        pltpu.VMEM_SHARED((NG, HID2), jnp.float32),
        pltpu.VMEM((NG, HID2), jnp.float32),
        pltpu.VMEM((VCH_FULL + 1, VC), jnp.int32),
        pltpu.VMEM(((VCH_FULL + 1) * VC, HID2), jnp.float32),
        pltpu.SemaphoreType.DMA,
    ],
)
def _sc_pool(h_h, batch_h, out, acc, zbuf, bslab, rslab, sem):
    c = lax.axis_index("c")
    s = lax.axis_index("s")
    w = _wid()
    voff = w * VCH_FULL + jnp.minimum(w, VCH_REM)
    vextra = w < VCH_REM

    @pl.when(s == 0)
    def _():
        _zero_2d(zbuf, NG, HID2)
        pltpu.sync_copy(zbuf, acc)

    pltpu.sync_copy(batch_h.at[pl.ds(voff, VCH_FULL)],
                    bslab.at[pl.ds(0, VCH_FULL)])
    pltpu.sync_copy(h_h.at[pl.ds(voff * VC, VCH_FULL * VC)],
                    rslab.at[pl.ds(0, VCH_FULL * VC)])

    @pl.when(vextra)
    def _():
        pltpu.sync_copy(batch_h.at[pl.ds(voff + VCH_FULL, 1)],
                        bslab.at[pl.ds(VCH_FULL, 1)])
        pltpu.sync_copy(h_h.at[pl.ds((voff + VCH_FULL) * VC, VC)],
                        rslab.at[pl.ds(VCH_FULL * VC, VC)])

    plsc.subcore_barrier()

    def fire(t):
        return pltpu.async_copy(rslab.at[pl.ds(t * VC, VC)],
                                acc.at[bslab.at[t]], sem, add=True)

    descs = [fire(t) for t in range(VCH_FULL)]
    for d in descs:
        d.wait()

    @pl.when(vextra)
    def _():
        fire(VCH_FULL).wait()

    plsc.subcore_barrier()

    @pl.when(s == 0)
    def _():
        pltpu.sync_copy(acc, out.at[c])


BR = 2000
GRID = N // BR


def _rows(cols):
    return pl.BlockSpec((BR, cols), lambda i: (i, 0))


def _prows(cols):
    return pl.BlockSpec((NC, BR, cols), lambda i: (0, i, 0))


def _full(shape):
    return pl.BlockSpec(shape, lambda i: (0,) * len(shape))


def _dot(a, b):
    return jnp.dot(a, b, preferred_element_type=jnp.float32)


def _tc_dis(deg_p):
    def body(dp, o):
        deg = dp[0] + dp[1]
        o[...] = jnp.where(deg > 0, lax.rsqrt(deg), 0.0)

    return pl.pallas_call(
        body,
        out_shape=jax.ShapeDtypeStruct((NPAD,), jnp.float32),
    )(deg_p)


def _tc_l1pre(x, W1):
    def body(xr, wr, a0, a1, a2):
        a0[...] = _dot(xr[...], wr[0])
        a1[...] = _dot(xr[...], wr[1])
        a2[...] = _dot(xr[...], wr[2])

    o = jax.ShapeDtypeStruct((N, HID1), jnp.float32)
    return pl.pallas_call(
        body,
        grid=(GRID,),
        in_specs=[_rows(F), _full((3, F, HID1))],
        out_specs=[_rows(HID1)] * 3,
        out_shape=[o, o, o],
    )(x, W1)


def _tc_comb(a, p, scale):
    def body(ar, pr, o):
        o[...] = ar[...] + scale * (pr[0] + pr[1])

    return pl.pallas_call(
        body,
        grid=(GRID,),
        in_specs=[_rows(a.shape[1]), _prows(a.shape[1])],
        out_specs=_rows(a.shape[1]),
        out_shape=jax.ShapeDtypeStruct(a.shape, jnp.float32),
    )(a, p)


def _tc_l1post(a0, a2, q, b1):
    def body(a0r, a2r, qr, br, o):
        o[...] = jnp.maximum(a0r[...] - a2r[...] + qr[0] + qr[1] + br[...], 0.0)

    return pl.pallas_call(
        body,
        grid=(GRID,),
        in_specs=[_rows(HID1), _rows(HID1), _prows(HID1), _full((HID1,))],
        out_specs=_rows(HID1),
        out_shape=jax.ShapeDtypeStruct((N, HID1), jnp.float32),
    )(a0, a2, q, b1)


def _tc_mid(h, p, W):
    dout = W.shape[2]

    def body(hr, pr, wr, t1, z):
        t = pr[0] + pr[1]
        t1[...] = t
        z[...] = _dot(hr[...], wr[0]) + _dot(t, wr[1])

    return pl.pallas_call(
        body,
        grid=(GRID,),
        in_specs=[_rows(HID1), _prows(HID1), _full(W.shape)],
        out_specs=[_rows(HID1), _rows(dout)],
        out_shape=[
            jax.ShapeDtypeStruct((N, HID1), jnp.float32),
            jax.ShapeDtypeStruct((N, dout), jnp.float32),
        ],
    )(h, p, W)


def _tc_post(z, q, h, W2k, b):
    dout = W2k.shape[1]

    def body(zr, qr, hr, wr, br, o):
        u = 2.0 * (qr[0] + qr[1]) - hr[...]
        o[...] = jnp.maximum(zr[...] + _dot(u, wr[...]) + br[...], 0.0)

    return pl.pallas_call(
        body,
        grid=(GRID,),
        in_specs=[_rows(dout), _prows(HID1), _rows(HID1), _full(W2k.shape),
                  _full(b.shape)],
        out_specs=_rows(dout),
        out_shape=jax.ShapeDtypeStruct((N, dout), jnp.float32),
    )(z, q, h, W2k, b)


def _tc_head(sums, cnt_p, Wl, bl):
    def body(sr, cr, wr, br, o):
        cnt = jnp.maximum(cr[0] + cr[1], 1.0)
        pooled = (sr[0] + sr[1]) / cnt[:, None]
        logits = _dot(pooled, wr[...]) + br[...]
        m = jnp.max(logits, axis=1, keepdims=True)
        lse = m + jnp.log(jnp.sum(jnp.exp(logits - m), axis=1, keepdims=True))
        o[...] = logits - lse

    return pl.pallas_call(
        body,
        out_shape=jax.ShapeDtypeStruct((NG, NCLS), jnp.float32),
    )(sums, cnt_p, Wl, bl)


def kernel(x, edge_index, edge_attr, batch, W1, b1, W2, b2, W3, b3, Wl, bl):
    src = edge_index[0]
    dst = edge_index[1]
    src2 = src.reshape(NECH, EC)
    dst2 = dst.reshape(NECH, EC)

    batch2 = batch.reshape(NVCH, VC)
    deg_p, cnt_p = _sc_pre(src2, edge_attr.reshape(NECH, EC), batch2)
    dis = _tc_dis(deg_p)
    w = _sc_w(dis, src2, dst2, edge_attr.reshape(NECH, EC))

    a0, a1, a2 = _tc_l1pre(x, W1)
    p = _sc_prop(a2, src2, dst2, w)
    y = _tc_comb(a1, p, 2.0)
    q = _sc_prop(y, src2, dst2, w)
    h1 = _tc_l1post(a0, a2, q, b1)

    p = _sc_prop(h1, src2, dst2, w)
    t1, z = _tc_mid(h1, p, W2)
    q = _sc_prop(t1, src2, dst2, w)
    h2 = _tc_post(z, q, h1, W2[2], b2)

    p = _sc_prop(h2, src2, dst2, w)
    t1, z = _tc_mid(h2, p, W3)
    q = _sc_prop(t1, src2, dst2, w)
    h3 = _tc_post(z, q, h2, W3[2], b3)

    sums = _sc_pool(h3, batch2)
    return _tc_head(sums, cnt_p, Wl, bl)


# speedup vs baseline: 1.9222x; 1.7800x over previous
import functools

import jax
import jax.numpy as jnp
from jax import lax
from jax.experimental import pallas as pl
from jax.experimental.pallas import tpu as pltpu
from jax.experimental.pallas import tpu_sc as plsc

N, E, F, HID1, HID2, NCLS, NG = 10000, 320000, 128, 64, 128, 40, 64
NPAD = 10240
NC, NS, L = 2, 16, 16
NW = NC * NS
EC = 128
NECH = E // EC
ECH_FULL = NECH // NW
ECH_REM = NECH % NW
VC = 80
NVCH = N // VC
VCH_FULL = NVCH // NW
VCH_REM = NVCH % NW

@functools.cache
def _mesh():
    return plsc.VectorSubcoreMesh(core_axis_name="c", subcore_axis_name="s",
                                  num_cores=NC, num_subcores=NS)


def _sc(out_type, scratch_types):
    def deco(body):
        @functools.cache
        def build():
            return pl.kernel(
                body, out_type, mesh=_mesh(), scratch_types=scratch_types,
                compiler_params=pltpu.CompilerParams(
                    needs_layout_passes=False, use_tc_tiling_on_sc=False))

        def call(*args):
            return build()(*args)

        return call

    return deco


def _wid():
    return lax.axis_index("s") * NC + lax.axis_index("c")


def _zero_1d(ref, nwords):
    z = jnp.zeros((L,), jnp.float32)

    def body(i, _):
        ref[pl.ds(i * L, L)] = z
        return 0

    lax.fori_loop(0, nwords // L, body, 0)


def _zero_2d(ref, rows, cols):
    z = jnp.zeros((L,), jnp.float32)

    def body(i, _):
        for j in range(cols // L):
            ref[i, pl.ds(j * L, L)] = z
        return 0

    lax.fori_loop(0, rows, body, 0)


FIRE = 8
EGRP = ECH_FULL // FIRE
ETAIL = ECH_FULL - EGRP * FIRE


@_sc(
    out_type=(
        jax.ShapeDtypeStruct((NC, NPAD), jnp.float32),
        jax.ShapeDtypeStruct((NC, NG), jnp.float32),
    ),
    scratch_types=[
        pltpu.VMEM_SHARED((NPAD,), jnp.float32),
        pltpu.VMEM_SHARED((NG,), jnp.float32),
        pltpu.VMEM((640,), jnp.float32),
        pltpu.VMEM((ECH_FULL + 1, EC), jnp.int32),
        pltpu.VMEM((ECH_FULL + 1, EC), jnp.float32),
        pltpu.VMEM((VCH_FULL + 1, VC), jnp.int32),
        pltpu.VMEM((VC,), jnp.float32),
        pltpu.SemaphoreType.DMA,
    ],
)
def _sc_pre(src_h, ea_h, batch_h, deg_out, cnt_out, deg_acc, cnt_acc, zbuf,
            sslab, easlab, bslab, ones, sem):
    c = lax.axis_index("c")
    s = lax.axis_index("s")
    w = _wid()
    row_off = w * ECH_FULL + jnp.minimum(w, ECH_REM)
    extra = w < ECH_REM
    voff = w * VCH_FULL + jnp.minimum(w, VCH_REM)
    vextra = w < VCH_REM

    _zero_1d(zbuf, 640)
    pltpu.sync_copy(zbuf, deg_acc.at[pl.ds(s * 640, 640)])

    @pl.when(s == 0)
    def _():
        pltpu.sync_copy(zbuf.at[pl.ds(0, NG)], cnt_acc)

    one = jnp.ones((L,), jnp.float32)
    for j in range(VC // L):
        ones[pl.ds(j * L, L)] = one

    pltpu.sync_copy(src_h.at[pl.ds(row_off, ECH_FULL)],
                    sslab.at[pl.ds(0, ECH_FULL)])
    pltpu.sync_copy(ea_h.at[pl.ds(row_off, ECH_FULL)],
                    easlab.at[pl.ds(0, ECH_FULL)])
    pltpu.sync_copy(batch_h.at[pl.ds(voff, VCH_FULL)],
                    bslab.at[pl.ds(0, VCH_FULL)])

    @pl.when(extra)
    def _():
        pltpu.sync_copy(src_h.at[pl.ds(row_off + ECH_FULL, 1)],
                        sslab.at[pl.ds(ECH_FULL, 1)])
        pltpu.sync_copy(ea_h.at[pl.ds(row_off + ECH_FULL, 1)],
                        easlab.at[pl.ds(ECH_FULL, 1)])

    @pl.when(vextra)
    def _():
        pltpu.sync_copy(batch_h.at[pl.ds(voff + VCH_FULL, 1)],
                        bslab.at[pl.ds(VCH_FULL, 1)])

    plsc.subcore_barrier()

    def fire(t):
        return pltpu.async_copy(easlab.at[t], deg_acc.at[sslab.at[t]], sem,
                                add=True)

    def group(k, _):
        descs = [fire(FIRE * k + b) for b in range(FIRE)]
        for d in descs:
            d.wait()
        return 0

    lax.fori_loop(0, EGRP, group, 0)
    descs = [fire(EGRP * FIRE + b) for b in range(ETAIL)]
    for d in descs:
        d.wait()

    @pl.when(extra)
    def _():
        fire(ECH_FULL).wait()

    def vfire(t):
        return pltpu.async_copy(ones, cnt_acc.at[bslab.at[t]], sem, add=True)

    descs = [vfire(t) for t in range(VCH_FULL)]
    for d in descs:
        d.wait()

    @pl.when(vextra)
    def _():
        vfire(VCH_FULL).wait()

    plsc.subcore_barrier()
    pltpu.sync_copy(deg_acc.at[pl.ds(s * 640, 640)],
                    deg_out.at[c, pl.ds(s * 640, 640)])

    @pl.when(s == 0)
    def _():
        pltpu.sync_copy(cnt_acc, cnt_out.at[c])


@_sc(
    out_type=jax.ShapeDtypeStruct((NECH, EC), jnp.float32),
    scratch_types=[
        pltpu.VMEM((NPAD,), jnp.float32),
        pltpu.VMEM((ECH_FULL + 1, EC), jnp.int32),
        pltpu.VMEM((ECH_FULL + 1, EC), jnp.int32),
        pltpu.VMEM((ECH_FULL + 1, EC), jnp.float32),
        pltpu.VMEM((ECH_FULL + 1, EC), jnp.float32),
    ],
)
def _sc_w(dis_h, src_h, dst_h, ea_h, w_out, dis_v, sslab, dslab, easlab, wslab):
    w = _wid()
    row_off = w * ECH_FULL + jnp.minimum(w, ECH_REM)
    extra = w < ECH_REM
    pltpu.sync_copy(dis_h, dis_v)
    pltpu.sync_copy(src_h.at[pl.ds(row_off, ECH_FULL)],
                    sslab.at[pl.ds(0, ECH_FULL)])
    pltpu.sync_copy(dst_h.at[pl.ds(row_off, ECH_FULL)],
                    dslab.at[pl.ds(0, ECH_FULL)])
    pltpu.sync_copy(ea_h.at[pl.ds(row_off, ECH_FULL)],
                    easlab.at[pl.ds(0, ECH_FULL)])

    @pl.when(extra)
    def _():
        pltpu.sync_copy(src_h.at[pl.ds(row_off + ECH_FULL, 1)],
                        sslab.at[pl.ds(ECH_FULL, 1)])
        pltpu.sync_copy(dst_h.at[pl.ds(row_off + ECH_FULL, 1)],
                        dslab.at[pl.ds(ECH_FULL, 1)])
        pltpu.sync_copy(ea_h.at[pl.ds(row_off + ECH_FULL, 1)],
                        easlab.at[pl.ds(ECH_FULL, 1)])

    def body(t, _):
        for g in range(EC // L):
            isrc = sslab[t, pl.ds(g * L, L)]
            idst = dslab[t, pl.ds(g * L, L)]
            dsrc = plsc.load_gather(dis_v, [isrc])
            ddst = plsc.load_gather(dis_v, [idst])
            wslab[t, pl.ds(g * L, L)] = -(
                dsrc * easlab[t, pl.ds(g * L, L)] * ddst)
        return 0

    lax.fori_loop(0, ECH_FULL, body, 0)

    @pl.when(extra)
    def _():
        body(ECH_FULL, 0)

    pltpu.sync_copy(wslab.at[pl.ds(0, ECH_FULL)],
                    w_out.at[pl.ds(row_off, ECH_FULL)])

    @pl.when(extra)
    def _():
        pltpu.sync_copy(wslab.at[pl.ds(ECH_FULL, 1)],
                        w_out.at[pl.ds(row_off + ECH_FULL, 1)])


NBUF = 6
GRPS = ECH_FULL // NBUF
HALF = NBUF // 2


@_sc(
    out_type=jax.ShapeDtypeStruct((NC, NPAD, HID1), jnp.float32),
    scratch_types=[
        pltpu.VMEM_SHARED((NPAD, HID1), jnp.float32),
        pltpu.VMEM((ECH_FULL + 1, EC), jnp.int32),
        pltpu.VMEM((ECH_FULL + 1, EC), jnp.int32),
        pltpu.VMEM((ECH_FULL + 1, EC), jnp.float32),
    ]
    + [pltpu.VMEM((EC, HID1), jnp.float32)] * NBUF
    + [pltpu.SemaphoreType.DMA] * (2 * NBUF),
)
def _sc_prop(h_h, src_h, dst_h, w_h, out, acc, sslab, dslab, wslab,
             r0, r1, r2, r3, r4, r5,
             g0, g1, g2, g3, g4, g5,
             s0, s1, s2, s3, s4, s5):
    c = lax.axis_index("c")
    s = lax.axis_index("s")
    w = _wid()
    rows = [r0, r1, r2, r3, r4, r5]
    gsem = [g0, g1, g2, g3, g4, g5]
    ssem = [s0, s1, s2, s3, s4, s5]
    row_off = w * ECH_FULL + jnp.minimum(w, ECH_REM)
    extra = w < ECH_REM

    _zero_2d(r0, EC, HID1)
    for k in range(5):
        pltpu.sync_copy(r0, acc.at[pl.ds(s * 640 + k * EC, EC)])

    pltpu.sync_copy(src_h.at[pl.ds(row_off, ECH_FULL)],
                    sslab.at[pl.ds(0, ECH_FULL)])
    pltpu.sync_copy(dst_h.at[pl.ds(row_off, ECH_FULL)],
                    dslab.at[pl.ds(0, ECH_FULL)])
    pltpu.sync_copy(w_h.at[pl.ds(row_off, ECH_FULL)],
                    wslab.at[pl.ds(0, ECH_FULL)])

    @pl.when(extra)
    def _():
        pltpu.sync_copy(src_h.at[pl.ds(row_off + ECH_FULL, 1)],
                        sslab.at[pl.ds(ECH_FULL, 1)])
        pltpu.sync_copy(dst_h.at[pl.ds(row_off + ECH_FULL, 1)],
                        dslab.at[pl.ds(ECH_FULL, 1)])
        pltpu.sync_copy(w_h.at[pl.ds(row_off + ECH_FULL, 1)],
                        wslab.at[pl.ds(ECH_FULL, 1)])

    plsc.subcore_barrier()

    def gather(t, b):
        return pltpu.async_copy(h_h.at[sslab.at[t]], rows[b], gsem[b])

    def wait_gather(b):
        pltpu.make_async_copy(h_h.at[sslab.at[0]], rows[b], gsem[b]).wait()

    def scatter(t, b):
        pltpu.async_copy(rows[b], acc.at[dslab.at[t]], ssem[b], add=True)

    def wait_scatter(b):
        pltpu.make_async_copy(rows[b], acc.at[dslab.at[0]], ssem[b]).wait()

    def mul(t, rbuf):
        tsp = jnp.full((L,), t, jnp.int32)

        @functools.partial(plsc.parallel_loop, 0, EC, unroll=8)
        def _(i):
            sp = plsc.load_gather(wslab, [tsp, jnp.full((L,), i, jnp.int32)])
            for j in range(HID1 // L):
                rbuf[i, pl.ds(j * L, L)] = rbuf[i, pl.ds(j * L, L)] * sp

    def process(t, b):
        wait_gather(b)
        mul(t, rows[b])
        scatter(t, b)

    for b in range(HALF):
        gather(b, b)

    def group(k, _):
        for b in range(HALF):
            t = NBUF * k + HALF + b

            @pl.when(k > 0)
            def _(b=b):
                wait_scatter(HALF + b)

            gather(t, HALF + b)
        for b in range(HALF):
            process(NBUF * k + b, b)
        for b in range(HALF):
            t = NBUF * k + NBUF + b

            @pl.when(k < GRPS - 1)
            def _(b=b, t=t):
                wait_scatter(b)
                gather(t, b)

        for b in range(HALF):
            process(NBUF * k + HALF + b, HALF + b)
        return 0

    lax.fori_loop(0, GRPS, group, 0)

    @pl.when(extra)
    def _():
        wait_scatter(0)
        gather(ECH_FULL, 0)
        process(ECH_FULL, 0)

    for b in range(NBUF):
        wait_scatter(b)

    plsc.subcore_barrier()
    pltpu.sync_copy(acc.at[pl.ds(s * 640, 640)], out.at[c, pl.ds(s * 640, 640)])


@_sc(
    out_type=jax.ShapeDtypeStruct((NC, NG, HID2), jnp.float32),
    scratch_types=[
        pltpu.VMEM_SHARED((NG, HID2), jnp.float32),
        pltpu.VMEM((NG, HID2), jnp.float32),
        pltpu.VMEM((VCH_FULL + 1, VC), jnp.int32),
        pltpu.VMEM(((VCH_FULL + 1) * VC, HID2), jnp.float32),
        pltpu.SemaphoreType.DMA,
    ],
)
def _sc_pool(h_h, batch_h, out, acc, zbuf, bslab, rslab, sem):
    c = lax.axis_index("c")
    s = lax.axis_index("s")
    w = _wid()
    voff = w * VCH_FULL + jnp.minimum(w, VCH_REM)
    vextra = w < VCH_REM

    @pl.when(s == 0)
    def _():
        _zero_2d(zbuf, NG, HID2)
        pltpu.sync_copy(zbuf, acc)

    pltpu.sync_copy(batch_h.at[pl.ds(voff, VCH_FULL)],
                    bslab.at[pl.ds(0, VCH_FULL)])
    pltpu.sync_copy(h_h.at[pl.ds(voff * VC, VCH_FULL * VC)],
                    rslab.at[pl.ds(0, VCH_FULL * VC)])

    @pl.when(vextra)
    def _():
        pltpu.sync_copy(batch_h.at[pl.ds(voff + VCH_FULL, 1)],
                        bslab.at[pl.ds(VCH_FULL, 1)])
        pltpu.sync_copy(h_h.at[pl.ds((voff + VCH_FULL) * VC, VC)],
                        rslab.at[pl.ds(VCH_FULL * VC, VC)])

    plsc.subcore_barrier()

    def fire(t):
        return pltpu.async_copy(rslab.at[pl.ds(t * VC, VC)],
                                acc.at[bslab.at[t]], sem, add=True)

    descs = [fire(t) for t in range(VCH_FULL)]
    for d in descs:
        d.wait()

    @pl.when(vextra)
    def _():
        fire(VCH_FULL).wait()

    plsc.subcore_barrier()

    @pl.when(s == 0)
    def _():
        pltpu.sync_copy(acc, out.at[c])


BR = 2000
GRID = N // BR


def _rows(cols):
    return pl.BlockSpec((BR, cols), lambda i: (i, 0))


def _prows(cols):
    return pl.BlockSpec((NC, BR, cols), lambda i: (0, i, 0))


def _full(shape):
    return pl.BlockSpec(shape, lambda i: (0,) * len(shape))


def _dot(a, b):
    return jnp.dot(a, b, preferred_element_type=jnp.float32)


def _tc_dis(deg_p):
    def body(dp, o):
        deg = dp[0] + dp[1]
        o[...] = jnp.where(deg > 0, lax.rsqrt(deg), 0.0)

    return pl.pallas_call(
        body,
        out_shape=jax.ShapeDtypeStruct((NPAD,), jnp.float32),
    )(deg_p)


def _tc_l1pre(x, W1):
    def body(xr, wr, a0, a1, a2):
        a0[...] = _dot(xr[...], wr[0])
        a1[...] = _dot(xr[...], wr[1])
        a2[...] = _dot(xr[...], wr[2])

    o = jax.ShapeDtypeStruct((N, HID1), jnp.float32)
    return pl.pallas_call(
        body,
        grid=(GRID,),
        in_specs=[_rows(F), _full((3, F, HID1))],
        out_specs=[_rows(HID1)] * 3,
        out_shape=[o, o, o],
    )(x, W1)


def _tc_comb(a, p, scale):
    def body(ar, pr, o):
        o[...] = ar[...] + scale * (pr[0] + pr[1])

    return pl.pallas_call(
        body,
        grid=(GRID,),
        in_specs=[_rows(a.shape[1]), _prows(a.shape[1])],
        out_specs=_rows(a.shape[1]),
        out_shape=jax.ShapeDtypeStruct(a.shape, jnp.float32),
    )(a, p)


def _tc_l1post(a0, a2, q, b1):
    def body(a0r, a2r, qr, br, o):
        o[...] = jnp.maximum(a0r[...] - a2r[...] + qr[0] + qr[1] + br[...], 0.0)

    return pl.pallas_call(
        body,
        grid=(GRID,),
        in_specs=[_rows(HID1), _rows(HID1), _prows(HID1), _full((HID1,))],
        out_specs=_rows(HID1),
        out_shape=jax.ShapeDtypeStruct((N, HID1), jnp.float32),
    )(a0, a2, q, b1)


def _tc_mid(h, p, W):
    dout = W.shape[2]

    def body(hr, pr, wr, t1, z):
        t = pr[0] + pr[1]
        t1[...] = t
        z[...] = _dot(hr[...], wr[0]) + _dot(t, wr[1])

    return pl.pallas_call(
        body,
        grid=(GRID,),
        in_specs=[_rows(HID1), _prows(HID1), _full(W.shape)],
        out_specs=[_rows(HID1), _rows(dout)],
        out_shape=[
            jax.ShapeDtypeStruct((N, HID1), jnp.float32),
            jax.ShapeDtypeStruct((N, dout), jnp.float32),
        ],
    )(h, p, W)


def _tc_post(z, q, h, W2k, b):
    dout = W2k.shape[1]

    def body(zr, qr, hr, wr, br, o):
        u = 2.0 * (qr[0] + qr[1]) - hr[...]
        o[...] = jnp.maximum(zr[...] + _dot(u, wr[...]) + br[...], 0.0)

    return pl.pallas_call(
        body,
        grid=(GRID,),
        in_specs=[_rows(dout), _prows(HID1), _rows(HID1), _full(W2k.shape),
                  _full(b.shape)],
        out_specs=_rows(dout),
        out_shape=jax.ShapeDtypeStruct((N, dout), jnp.float32),
    )(z, q, h, W2k, b)


def _tc_head(sums, cnt_p, Wl, bl):
    def body(sr, cr, wr, br, o):
        cnt = jnp.maximum(cr[0] + cr[1], 1.0)
        pooled = (sr[0] + sr[1]) / cnt[:, None]
        logits = _dot(pooled, wr[...]) + br[...]
        m = jnp.max(logits, axis=1, keepdims=True)
        lse = m + jnp.log(jnp.sum(jnp.exp(logits - m), axis=1, keepdims=True))
        o[...] = logits - lse

    return pl.pallas_call(
        body,
        out_shape=jax.ShapeDtypeStruct((NG, NCLS), jnp.float32),
    )(sums, cnt_p, Wl, bl)


def kernel(x, edge_index, edge_attr, batch, W1, b1, W2, b2, W3, b3, Wl, bl):
    src = edge_index[0]
    dst = edge_index[1]
    src2 = src.reshape(NECH, EC)
    dst2 = dst.reshape(NECH, EC)

    batch2 = batch.reshape(NVCH, VC)
    deg_p, cnt_p = _sc_pre(src2, edge_attr.reshape(NECH, EC), batch2)
    dis = _tc_dis(deg_p)
    w = _sc_w(dis, src2, dst2, edge_attr.reshape(NECH, EC))

    a0, a1, a2 = _tc_l1pre(x, W1)
    p = _sc_prop(a2, src2, dst2, w)
    y = _tc_comb(a1, p, 2.0)
    q = _sc_prop(y, src2, dst2, w)
    h1 = _tc_l1post(a0, a2, q, b1)

    p = _sc_prop(h1, src2, dst2, w)
    t1, z = _tc_mid(h1, p, W2)
    q = _sc_prop(t1, src2, dst2, w)
    h2 = _tc_post(z, q, h1, W2[2], b2)

    p = _sc_prop(h2, src2, dst2, w)
    t1, z = _tc_mid(h2, p, W3)
    q = _sc_prop(t1, src2, dst2, w)
    h3 = _tc_post(z, q, h2, W3[2], b3)

    sums = _sc_pool(h3, batch2)
    return _tc_head(sums, cnt_p, Wl, bl)
